# baseline (speedup 1.0000x reference)
import functools

import jax
import jax.numpy as jnp
import numpy as np
from jax import lax
from jax.experimental import pallas as pl
from jax.experimental.pallas import tpu as pltpu
from jax.experimental.pallas import tpu_sc as plsc

HEADS = 4
CONV = 2

_NC, _NS = 2, 16
_CW = 16
_NCHUNK = 8
_NENT = 50000
_ACCR = 50048
_SBROWS = 16
_EBATCH = 128


def _seg_body(table, qinit, esrc, edst, out, acc, sidx, didx, rows0, rows1,
              sem_g0, sem_g1, sem_s0, sem_s1):
    core = lax.axis_index("c")
    sub = lax.axis_index("s")
    erows = esrc.shape[0] - _SBROWS
    rows_per_tile = erows // _NS
    n_super = rows_per_tile // _SBROWS
    acc_rows_per_tile = _ACCR // _NS
    rows_bufs = (rows0, rows1)
    sems_g = (sem_g0, sem_g1)
    sems_s = (sem_s0, sem_s1)

    for i in range(_NCHUNK // _NC):
        for c in range(_NC):
            chunk = _NC * i + c

            @pl.when(core == c)
            def _():
                r0 = sub * acc_rows_per_tile
                pltpu.sync_copy(qinit.at[chunk].at[pl.ds(r0, acc_rows_per_tile)],
                                acc.at[pl.ds(r0, acc_rows_per_tile)])

        plsc.subcore_barrier()

        for c in range(_NC):
            chunk = _NC * i + c

            @pl.when(core == c)
            def _():
                tab = table.at[chunk]

                def sb_step(b, carry):
                    base = sub * rows_per_tile + b * _SBROWS
                    pltpu.sync_copy(esrc.at[pl.ds(base, _SBROWS)], sidx)
                    pltpu.sync_copy(edst.at[pl.ds(base, _SBROWS)], didx)
                    g = [None, None]
                    s = [None, None]
                    g[0] = pltpu.async_copy(tab.at[sidx.at[0]], rows_bufs[0],
                                            sems_g[0])
                    for j in range(_SBROWS):
                        cb = j % 2
                        nb = (j + 1) % 2
                        g[cb].wait()
                        if j + 1 < _SBROWS:
                            if s[nb] is not None:
                                s[nb].wait()
                            g[nb] = pltpu.async_copy(tab.at[sidx.at[j + 1]],
                                                     rows_bufs[nb], sems_g[nb])
                        s[cb] = pltpu.async_copy(rows_bufs[cb],
                                                 acc.at[didx.at[j]],
                                                 sems_s[cb], add=True)
                    s[0].wait()
                    s[1].wait()
                    return carry

                lax.fori_loop(0, n_super, sb_step, 0)

        plsc.subcore_barrier()

        for c in range(_NC):
            chunk = _NC * i + c

            @pl.when(core == c)
            def _():
                r0 = sub * acc_rows_per_tile
                pltpu.sync_copy(acc.at[pl.ds(r0, acc_rows_per_tile)],
                                out.at[chunk].at[pl.ds(r0, acc_rows_per_tile)])

        plsc.subcore_barrier()


def _sc_segsum(table_chunks, qinit, esrc2d, edst2d):
    f = pl.kernel(
        _seg_body,
        out_type=jax.ShapeDtypeStruct((_NCHUNK, _ACCR, _CW), jnp.float32),
        mesh=plsc.VectorSubcoreMesh(core_axis_name="c", subcore_axis_name="s"),
        compiler_params=pltpu.CompilerParams(use_tc_tiling_on_sc=False),
        scratch_types=[
            pltpu.VMEM_SHARED((_ACCR, _CW), jnp.float32),
            pltpu.VMEM((_SBROWS, 128), jnp.int32),
            pltpu.VMEM((_SBROWS, 128), jnp.int32),
            pltpu.VMEM((_EBATCH, _CW), jnp.float32),
            pltpu.VMEM((_EBATCH, _CW), jnp.float32),
            pltpu.SemaphoreType.DMA,
            pltpu.SemaphoreType.DMA,
            pltpu.SemaphoreType.DMA,
            pltpu.SemaphoreType.DMA,
        ],
    )
    return f(table_chunks, qinit, esrc2d, edst2d)


def _pad_edges(src, dst, dummy_dst):
    e = src.shape[0]
    unit = _NS * 2 * _SBROWS * 128
    epad = ((e + unit - 1) // unit) * unit
    pad = epad - e + _SBROWS * 128
    src = jnp.concatenate([src.astype(jnp.int32), jnp.zeros((pad,), jnp.int32)])
    dst = jnp.concatenate([dst.astype(jnp.int32),
                           jnp.full((pad,), dummy_dst, jnp.int32)])
    return src.reshape(-1, 128), dst.reshape(-1, 128)


def _mhsa_mean(x, Wq, Wk, Wv):
    B, S, D = x.shape
    h = D // HEADS
    q = (x @ Wq).reshape(B, S, HEADS, h).transpose(0, 2, 1, 3)
    k = (x @ Wk).reshape(B, S, HEADS, h).transpose(0, 2, 1, 3)
    v = (x @ Wv).reshape(B, S, HEADS, h).transpose(0, 2, 1, 3)
    att = jax.nn.softmax(jnp.einsum('bhqd,bhkd->bhqk', q, k) * (1.0 / float(np.sqrt(h))), axis=-1)
    return jnp.einsum('bhqk,bhkd->bhqd', att, v).transpose(0, 2, 1, 3).reshape(B, S, D).mean(axis=1)


def _score_body(pm_ref, cand_ref, out_ref):
    pm = pm_ref[...]
    cand = cand_ref[...]
    out_ref[...] = jnp.sum(pm[:, None, :] * cand, axis=-1)


def _chunked(x128):
    n = x128.shape[0]
    return jnp.moveaxis(x128.reshape(n, _NCHUNK, _CW), 1, 0)


def kernel(word_table, ent_table, d_Wq, d_Wk, d_Wv, t_Wq, t_Wk, t_Wv, W_t, b_t,
           users, items, query_words, neg_items, review_words, query_words_graph,
           profile_src, profile_dst, p_src, p_dst, q_id, pb_src, pb_dst):
    N, ED = ent_table.shape

    review_h = _mhsa_mean(word_table[review_words], d_Wq, d_Wk, d_Wv)
    deg_p = jnp.maximum(jax.ops.segment_sum(jnp.ones(profile_dst.shape[0], jnp.float32), profile_dst, N), 1.0)[:, None]
    entity_h = jax.ops.segment_sum(review_h[profile_src], profile_dst, N) / deg_p

    qw = word_table[query_words_graph]
    q_l = _mhsa_mean(qw, t_Wq, t_Wk, t_Wv) @ W_t + b_t
    q_h = _mhsa_mean(qw, d_Wq, d_Wk, d_Wv)
    q_e0 = jnp.concatenate([q_l, q_h], axis=-1)

    e0 = jnp.concatenate([ent_table, entity_h], axis=-1)
    deg_i = jnp.maximum(
        jax.ops.segment_sum(jnp.ones(p_src.shape[0], jnp.float32), p_src, N)
        + jax.ops.segment_sum(jnp.ones(pb_src.shape[0], jnp.float32), pb_src, N), 1.0)[:, None]
    inv_sd = jax.lax.rsqrt(deg_i)

    Q = jax.ops.segment_sum(q_e0[q_id] * inv_sd[p_src], p_dst, N)
    qinit = jnp.zeros((_NCHUNK, _ACCR, _CW), jnp.float32)
    qinit = qinit.at[:, :N, :].set(_chunked(Q))

    esrc2d, edst2d = _pad_edges(
        jnp.concatenate([p_src, pb_src + N]),
        jnp.concatenate([p_dst, pb_dst]), N)

    x = e0
    layers_sum = e0
    for _ in range(CONV):
        tab = _chunked(jnp.concatenate([x, x * inv_sd], axis=0))
        agg = _sc_segsum(tab, qinit, esrc2d, edst2d)
        agg = jnp.moveaxis(agg[:, :N, :], 0, 1).reshape(N, _NCHUNK * _CW)
        x = agg * inv_sd
        layers_sum = layers_sum + x
    ent_e = layers_sum * (1.0 / 3.0)

    user_emb = ent_e[users]
    qt = _mhsa_mean(word_table[query_words], t_Wq, t_Wk, t_Wv) @ W_t + b_t
    pm = user_emb[:, :ED] + qt
    cand = jnp.concatenate([ent_e[items][:, None, :ED], ent_e[neg_items][:, :, :ED]], axis=1)

    B = pm.shape[0]
    out = pl.pallas_call(
        _score_body,
        out_shape=jax.ShapeDtypeStruct((B, 6), jnp.float32),
    )(pm, cand)
    return out


# speedup vs baseline: 1.1695x; 1.1695x over previous
import functools

import jax
import jax.numpy as jnp
import numpy as np
from jax import lax
from jax.experimental import pallas as pl
from jax.experimental.pallas import tpu as pltpu
from jax.experimental.pallas import tpu_sc as plsc

HEADS = 4
CONV = 2

_NC, _NS = 2, 16
_CW = 16
_NCHUNK = 8
_NENT = 50000
_ACCR = 50048
_SBROWS = 16
_EBATCH = 128
_NBUF = 8


def _seg_body(table, qinit, esrc, edst, out, acc, sidx, didx, rows,
              sem_g, sem_s):
    core = lax.axis_index("c")
    sub = lax.axis_index("s")
    erows = esrc.shape[0] - _SBROWS
    rows_per_tile = erows // _NS
    n_super = rows_per_tile // _SBROWS
    acc_rows_per_tile = _ACCR // _NS

    for i in range(_NCHUNK // _NC):
        for c in range(_NC):
            chunk = _NC * i + c

            @pl.when(core == c)
            def _():
                r0 = sub * acc_rows_per_tile
                pltpu.sync_copy(qinit.at[chunk].at[pl.ds(r0, acc_rows_per_tile)],
                                acc.at[pl.ds(r0, acc_rows_per_tile)])

        plsc.subcore_barrier()

        for c in range(_NC):
            chunk = _NC * i + c

            @pl.when(core == c)
            def _():
                tab = table.at[chunk]

                def sb_step(b, carry):
                    base = sub * rows_per_tile + b * _SBROWS
                    pltpu.sync_copy(esrc.at[pl.ds(base, _SBROWS)], sidx)
                    pltpu.sync_copy(edst.at[pl.ds(base, _SBROWS)], didx)
                    g = [None] * _SBROWS
                    s = [None] * _SBROWS
                    for t in range(_SBROWS):
                        k = t % _NBUF
                        if t >= _NBUF:
                            s[t - _NBUF].wait()
                        g[t] = pltpu.async_copy(tab.at[sidx.at[t]],
                                                rows.at[k], sem_g.at[k])
                        r = t - (_NBUF - 1)
                        if r >= 0:
                            g[r].wait()
                            s[r] = pltpu.async_copy(rows.at[r % _NBUF],
                                                    acc.at[didx.at[r]],
                                                    sem_s.at[r % _NBUF],
                                                    add=True)
                    for r in range(_SBROWS - _NBUF + 1, _SBROWS):
                        g[r].wait()
                        s[r] = pltpu.async_copy(rows.at[r % _NBUF],
                                                acc.at[didx.at[r]],
                                                sem_s.at[r % _NBUF], add=True)
                    for r in range(_SBROWS - _NBUF, _SBROWS):
                        s[r].wait()
                    return carry

                lax.fori_loop(0, n_super, sb_step, 0)

        plsc.subcore_barrier()

        for c in range(_NC):
            chunk = _NC * i + c

            @pl.when(core == c)
            def _():
                r0 = sub * acc_rows_per_tile
                pltpu.sync_copy(acc.at[pl.ds(r0, acc_rows_per_tile)],
                                out.at[chunk].at[pl.ds(r0, acc_rows_per_tile)])

        plsc.subcore_barrier()


def _sc_segsum(table_chunks, qinit, esrc2d, edst2d):
    f = pl.kernel(
        _seg_body,
        out_type=jax.ShapeDtypeStruct((_NCHUNK, _ACCR, _CW), jnp.float32),
        mesh=plsc.VectorSubcoreMesh(core_axis_name="c", subcore_axis_name="s"),
        compiler_params=pltpu.CompilerParams(use_tc_tiling_on_sc=False),
        scratch_types=[
            pltpu.VMEM_SHARED((_ACCR, _CW), jnp.float32),
            pltpu.VMEM((_SBROWS, 128), jnp.int32),
            pltpu.VMEM((_SBROWS, 128), jnp.int32),
            pltpu.VMEM((_NBUF, _EBATCH, _CW), jnp.float32),
            pltpu.SemaphoreType.DMA((_NBUF,)),
            pltpu.SemaphoreType.DMA((_NBUF,)),
        ],
    )
    return f(table_chunks, qinit, esrc2d, edst2d)


def _pad_edges(src, dst, dummy_dst):
    e = src.shape[0]
    unit = _NS * 2 * _SBROWS * 128
    epad = ((e + unit - 1) // unit) * unit
    pad = epad - e + _SBROWS * 128
    src = jnp.concatenate([src.astype(jnp.int32), jnp.zeros((pad,), jnp.int32)])
    dst = jnp.concatenate([dst.astype(jnp.int32),
                           jnp.full((pad,), dummy_dst, jnp.int32)])
    return src.reshape(-1, 128), dst.reshape(-1, 128)


def _mhsa_mean(x, Wq, Wk, Wv):
    B, S, D = x.shape
    h = D // HEADS
    q = (x @ Wq).reshape(B, S, HEADS, h).transpose(0, 2, 1, 3)
    k = (x @ Wk).reshape(B, S, HEADS, h).transpose(0, 2, 1, 3)
    v = (x @ Wv).reshape(B, S, HEADS, h).transpose(0, 2, 1, 3)
    att = jax.nn.softmax(jnp.einsum('bhqd,bhkd->bhqk', q, k) * (1.0 / float(np.sqrt(h))), axis=-1)
    return jnp.einsum('bhqk,bhkd->bhqd', att, v).transpose(0, 2, 1, 3).reshape(B, S, D).mean(axis=1)


def _score_body(pm_ref, cand_ref, out_ref):
    pm = pm_ref[...]
    cand = cand_ref[...]
    out_ref[...] = jnp.sum(pm[:, None, :] * cand, axis=-1)


def _chunked(x128):
    n = x128.shape[0]
    return jnp.moveaxis(x128.reshape(n, _NCHUNK, _CW), 1, 0)


def kernel(word_table, ent_table, d_Wq, d_Wk, d_Wv, t_Wq, t_Wk, t_Wv, W_t, b_t,
           users, items, query_words, neg_items, review_words, query_words_graph,
           profile_src, profile_dst, p_src, p_dst, q_id, pb_src, pb_dst):
    N, ED = ent_table.shape

    review_h = _mhsa_mean(word_table[review_words], d_Wq, d_Wk, d_Wv)
    deg_p = jnp.maximum(jax.ops.segment_sum(jnp.ones(profile_dst.shape[0], jnp.float32), profile_dst, N), 1.0)[:, None]
    entity_h = jax.ops.segment_sum(review_h[profile_src], profile_dst, N) / deg_p

    qw = word_table[query_words_graph]
    q_l = _mhsa_mean(qw, t_Wq, t_Wk, t_Wv) @ W_t + b_t
    q_h = _mhsa_mean(qw, d_Wq, d_Wk, d_Wv)
    q_e0 = jnp.concatenate([q_l, q_h], axis=-1)

    e0 = jnp.concatenate([ent_table, entity_h], axis=-1)
    deg_i = jnp.maximum(
        jax.ops.segment_sum(jnp.ones(p_src.shape[0], jnp.float32), p_src, N)
        + jax.ops.segment_sum(jnp.ones(pb_src.shape[0], jnp.float32), pb_src, N), 1.0)[:, None]
    inv_sd = jax.lax.rsqrt(deg_i)

    Q = jax.ops.segment_sum(q_e0[q_id] * inv_sd[p_src], p_dst, N)
    qinit = jnp.zeros((_NCHUNK, _ACCR, _CW), jnp.float32)
    qinit = qinit.at[:, :N, :].set(_chunked(Q))

    esrc2d, edst2d = _pad_edges(
        jnp.concatenate([p_src, pb_src + N]),
        jnp.concatenate([p_dst, pb_dst]), N)

    x = e0
    layers_sum = e0
    for _ in range(CONV):
        tab = _chunked(jnp.concatenate([x, x * inv_sd], axis=0))
        agg = _sc_segsum(tab, qinit, esrc2d, edst2d)
        agg = jnp.moveaxis(agg[:, :N, :], 0, 1).reshape(N, _NCHUNK * _CW)
        x = agg * inv_sd
        layers_sum = layers_sum + x
    ent_e = layers_sum * (1.0 / 3.0)

    user_emb = ent_e[users]
    qt = _mhsa_mean(word_table[query_words], t_Wq, t_Wk, t_Wv) @ W_t + b_t
    pm = user_emb[:, :ED] + qt
    cand = jnp.concatenate([ent_e[items][:, None, :ED], ent_e[neg_items][:, :, :ED]], axis=1)

    B = pm.shape[0]
    out = pl.pallas_call(
        _score_body,
        out_shape=jax.ShapeDtypeStruct((B, 6), jnp.float32),
    )(pm, cand)
    return out


# speedup vs baseline: 1.2436x; 1.0634x over previous
import functools

import jax
import jax.numpy as jnp
import numpy as np
from jax import lax
from jax.experimental import pallas as pl
from jax.experimental.pallas import tpu as pltpu
from jax.experimental.pallas import tpu_sc as plsc

HEADS = 4
CONV = 2

_NC, _NS = 2, 16
_CW = 32
_NCHUNK = 4
_NENT = 50000
_ACCR = 50048
_SBROWS = 16
_EBATCH = 128
_NBUF = 4


def _seg_body(table, qinit, esrc, edst, out, acc, sidx, didx, rows,
              sem_g, sem_s):
    core = lax.axis_index("c")
    sub = lax.axis_index("s")
    erows = esrc.shape[0] - _SBROWS
    rows_per_tile = erows // _NS
    n_super = rows_per_tile // _SBROWS
    acc_rows_per_tile = _ACCR // _NS

    for i in range(_NCHUNK // _NC):
        for c in range(_NC):
            chunk = _NC * i + c

            @pl.when(core == c)
            def _():
                r0 = sub * acc_rows_per_tile
                pltpu.sync_copy(qinit.at[chunk].at[pl.ds(r0, acc_rows_per_tile)],
                                acc.at[pl.ds(r0, acc_rows_per_tile)])

        plsc.subcore_barrier()

        for c in range(_NC):
            chunk = _NC * i + c

            @pl.when(core == c)
            def _():
                tab = table.at[chunk]

                def sb_step(b, carry):
                    base = sub * rows_per_tile + b * _SBROWS
                    pltpu.sync_copy(esrc.at[pl.ds(base, _SBROWS)], sidx)
                    pltpu.sync_copy(edst.at[pl.ds(base, _SBROWS)], didx)
                    g = [None] * _SBROWS
                    s = [None] * _SBROWS
                    for t in range(_SBROWS):
                        k = t % _NBUF
                        if t >= _NBUF:
                            s[t - _NBUF].wait()
                        g[t] = pltpu.async_copy(tab.at[sidx.at[t]],
                                                rows.at[k], sem_g.at[k])
                        r = t - (_NBUF - 1)
                        if r >= 0:
                            g[r].wait()
                            s[r] = pltpu.async_copy(rows.at[r % _NBUF],
                                                    acc.at[didx.at[r]],
                                                    sem_s.at[r % _NBUF],
                                                    add=True)
                    for r in range(_SBROWS - _NBUF + 1, _SBROWS):
                        g[r].wait()
                        s[r] = pltpu.async_copy(rows.at[r % _NBUF],
                                                acc.at[didx.at[r]],
                                                sem_s.at[r % _NBUF], add=True)
                    for r in range(_SBROWS - _NBUF, _SBROWS):
                        s[r].wait()
                    return carry

                lax.fori_loop(0, n_super, sb_step, 0)

        plsc.subcore_barrier()

        for c in range(_NC):
            chunk = _NC * i + c

            @pl.when(core == c)
            def _():
                r0 = sub * acc_rows_per_tile
                pltpu.sync_copy(acc.at[pl.ds(r0, acc_rows_per_tile)],
                                out.at[chunk].at[pl.ds(r0, acc_rows_per_tile)])

        plsc.subcore_barrier()


def _sc_segsum(table_chunks, qinit, esrc2d, edst2d):
    f = pl.kernel(
        _seg_body,
        out_type=jax.ShapeDtypeStruct((_NCHUNK, _ACCR, _CW), jnp.float32),
        mesh=plsc.VectorSubcoreMesh(core_axis_name="c", subcore_axis_name="s"),
        compiler_params=pltpu.CompilerParams(use_tc_tiling_on_sc=False),
        scratch_types=[
            pltpu.VMEM_SHARED((_ACCR, _CW), jnp.float32),
            pltpu.VMEM((_SBROWS, 128), jnp.int32),
            pltpu.VMEM((_SBROWS, 128), jnp.int32),
            pltpu.VMEM((_NBUF, _EBATCH, _CW), jnp.float32),
            pltpu.SemaphoreType.DMA((_NBUF,)),
            pltpu.SemaphoreType.DMA((_NBUF,)),
        ],
    )
    return f(table_chunks, qinit, esrc2d, edst2d)


def _pad_edges(src, dst, dummy_dst):
    e = src.shape[0]
    unit = _NS * 2 * _SBROWS * 128
    epad = ((e + unit - 1) // unit) * unit
    pad = epad - e + _SBROWS * 128
    src = jnp.concatenate([src.astype(jnp.int32), jnp.zeros((pad,), jnp.int32)])
    dst = jnp.concatenate([dst.astype(jnp.int32),
                           jnp.full((pad,), dummy_dst, jnp.int32)])
    return src.reshape(-1, 128), dst.reshape(-1, 128)


def _mhsa_mean(x, Wq, Wk, Wv):
    B, S, D = x.shape
    h = D // HEADS
    q = (x @ Wq).reshape(B, S, HEADS, h).transpose(0, 2, 1, 3)
    k = (x @ Wk).reshape(B, S, HEADS, h).transpose(0, 2, 1, 3)
    v = (x @ Wv).reshape(B, S, HEADS, h).transpose(0, 2, 1, 3)
    att = jax.nn.softmax(jnp.einsum('bhqd,bhkd->bhqk', q, k) * (1.0 / float(np.sqrt(h))), axis=-1)
    return jnp.einsum('bhqk,bhkd->bhqd', att, v).transpose(0, 2, 1, 3).reshape(B, S, D).mean(axis=1)


def _score_body(pm_ref, cand_ref, out_ref):
    pm = pm_ref[...]
    cand = cand_ref[...]
    out_ref[...] = jnp.sum(pm[:, None, :] * cand, axis=-1)


def _chunked(x128):
    n = x128.shape[0]
    return jnp.moveaxis(x128.reshape(n, _NCHUNK, _CW), 1, 0)


def kernel(word_table, ent_table, d_Wq, d_Wk, d_Wv, t_Wq, t_Wk, t_Wv, W_t, b_t,
           users, items, query_words, neg_items, review_words, query_words_graph,
           profile_src, profile_dst, p_src, p_dst, q_id, pb_src, pb_dst):
    N, ED = ent_table.shape

    review_h = _mhsa_mean(word_table[review_words], d_Wq, d_Wk, d_Wv)
    deg_p = jnp.maximum(jax.ops.segment_sum(jnp.ones(profile_dst.shape[0], jnp.float32), profile_dst, N), 1.0)[:, None]
    entity_h = jax.ops.segment_sum(review_h[profile_src], profile_dst, N) / deg_p

    qw = word_table[query_words_graph]
    q_l = _mhsa_mean(qw, t_Wq, t_Wk, t_Wv) @ W_t + b_t
    q_h = _mhsa_mean(qw, d_Wq, d_Wk, d_Wv)
    q_e0 = jnp.concatenate([q_l, q_h], axis=-1)

    e0 = jnp.concatenate([ent_table, entity_h], axis=-1)
    deg_i = jnp.maximum(
        jax.ops.segment_sum(jnp.ones(p_src.shape[0], jnp.float32), p_src, N)
        + jax.ops.segment_sum(jnp.ones(pb_src.shape[0], jnp.float32), pb_src, N), 1.0)[:, None]
    inv_sd = jax.lax.rsqrt(deg_i)

    Q = jax.ops.segment_sum(q_e0[q_id] * inv_sd[p_src], p_dst, N)
    qinit = jnp.zeros((_NCHUNK, _ACCR, _CW), jnp.float32)
    qinit = qinit.at[:, :N, :].set(_chunked(Q))

    esrc2d, edst2d = _pad_edges(
        jnp.concatenate([p_src, pb_src + N]),
        jnp.concatenate([p_dst, pb_dst]), N)

    x = e0
    layers_sum = e0
    for _ in range(CONV):
        tab = _chunked(jnp.concatenate([x, x * inv_sd], axis=0))
        agg = _sc_segsum(tab, qinit, esrc2d, edst2d)
        agg = jnp.moveaxis(agg[:, :N, :], 0, 1).reshape(N, _NCHUNK * _CW)
        x = agg * inv_sd
        layers_sum = layers_sum + x
    ent_e = layers_sum * (1.0 / 3.0)

    user_emb = ent_e[users]
    qt = _mhsa_mean(word_table[query_words], t_Wq, t_Wk, t_Wv) @ W_t + b_t
    pm = user_emb[:, :ED] + qt
    cand = jnp.concatenate([ent_e[items][:, None, :ED], ent_e[neg_items][:, :, :ED]], axis=1)

    B = pm.shape[0]
    out = pl.pallas_call(
        _score_body,
        out_shape=jax.ShapeDtypeStruct((B, 6), jnp.float32),
    )(pm, cand)
    return out


# speedup vs baseline: 2.2294x; 1.7927x over previous
import functools

import jax
import jax.numpy as jnp
import numpy as np
from jax import lax
from jax.experimental import pallas as pl
from jax.experimental.pallas import tpu as pltpu
from jax.experimental.pallas import tpu_sc as plsc

HEADS = 4
CONV = 2

_NC, _NS = 2, 16
_CW = 32
_NCHUNK = 4
_NENT = 50000
_ACCR = 50048
_SBROWS = 16
_EBATCH = 128
_NBUF = 4


def _seg_body(table, qinit, esrc, edst, out, acc, sidx, didx, rows,
              sem_g, sem_s):
    core = lax.axis_index("c")
    sub = lax.axis_index("s")
    nchunk = table.shape[0]
    erows = esrc.shape[0] - _SBROWS
    rows_per_tile = erows // _NS
    n_super = rows_per_tile // _SBROWS
    acc_rows_per_tile = _ACCR // _NS

    for i in range(nchunk // _NC):
        for c in range(_NC):
            chunk = _NC * i + c

            @pl.when(core == c)
            def _():
                r0 = sub * acc_rows_per_tile
                pltpu.sync_copy(qinit.at[chunk].at[pl.ds(r0, acc_rows_per_tile)],
                                acc.at[pl.ds(r0, acc_rows_per_tile)])

        plsc.subcore_barrier()

        for c in range(_NC):
            chunk = _NC * i + c

            @pl.when(core == c)
            def _():
                tab = table.at[chunk]

                def sb_step(b, carry):
                    base = sub * rows_per_tile + b * _SBROWS
                    pltpu.sync_copy(esrc.at[pl.ds(base, _SBROWS)], sidx)
                    pltpu.sync_copy(edst.at[pl.ds(base, _SBROWS)], didx)
                    g = [None] * _SBROWS
                    s = [None] * _SBROWS
                    for t in range(_SBROWS):
                        k = t % _NBUF
                        if t >= _NBUF:
                            s[t - _NBUF].wait()
                        g[t] = pltpu.async_copy(tab.at[sidx.at[t]],
                                                rows.at[k], sem_g.at[k])
                        r = t - (_NBUF - 1)
                        if r >= 0:
                            g[r].wait()
                            s[r] = pltpu.async_copy(rows.at[r % _NBUF],
                                                    acc.at[didx.at[r]],
                                                    sem_s.at[r % _NBUF],
                                                    add=True)
                    for r in range(_SBROWS - _NBUF + 1, _SBROWS):
                        g[r].wait()
                        s[r] = pltpu.async_copy(rows.at[r % _NBUF],
                                                acc.at[didx.at[r]],
                                                sem_s.at[r % _NBUF], add=True)
                    for r in range(_SBROWS - _NBUF, _SBROWS):
                        s[r].wait()
                    return carry

                lax.fori_loop(0, n_super, sb_step, 0)

        plsc.subcore_barrier()

        for c in range(_NC):
            chunk = _NC * i + c

            @pl.when(core == c)
            def _():
                r0 = sub * acc_rows_per_tile
                pltpu.sync_copy(acc.at[pl.ds(r0, acc_rows_per_tile)],
                                out.at[chunk].at[pl.ds(r0, acc_rows_per_tile)])

        plsc.subcore_barrier()


def _sc_segsum(table_chunks, qinit, esrc2d, edst2d):
    nchunk = table_chunks.shape[0]
    f = pl.kernel(
        _seg_body,
        out_type=jax.ShapeDtypeStruct((nchunk, _ACCR, _CW), jnp.float32),
        mesh=plsc.VectorSubcoreMesh(core_axis_name="c", subcore_axis_name="s"),
        compiler_params=pltpu.CompilerParams(use_tc_tiling_on_sc=False),
        scratch_types=[
            pltpu.VMEM_SHARED((_ACCR, _CW), jnp.float32),
            pltpu.VMEM((_SBROWS, 128), jnp.int32),
            pltpu.VMEM((_SBROWS, 128), jnp.int32),
            pltpu.VMEM((_NBUF, _EBATCH, _CW), jnp.float32),
            pltpu.SemaphoreType.DMA((_NBUF,)),
            pltpu.SemaphoreType.DMA((_NBUF,)),
        ],
    )
    return f(table_chunks, qinit, esrc2d, edst2d)


def _q_body(qtab, invsd, qinit, esrc, escl, edst, out, acc, sidx, widx, didx,
            rows, wbuf, sem_g, sem_w, sem_s):
    core = lax.axis_index("c")
    sub = lax.axis_index("s")
    nchunk = qtab.shape[0]
    erows = esrc.shape[0] - _SBROWS
    rows_per_tile = erows // _NS
    n_super = rows_per_tile // _SBROWS
    acc_rows_per_tile = _ACCR // _NS

    def scale_rows(k):
        def srow(gi, carry):
            wv = wbuf[k, pl.ds(16 * gi, 16)]
            for u in range(16):
                ws = jnp.full((16,), wv[u], jnp.float32)
                for half in range(_CW // 16):
                    sl = pl.ds(16 * half, 16)
                    rows[k, 16 * gi + u, sl] = rows[k, 16 * gi + u, sl] * ws
            return carry

        lax.fori_loop(0, _EBATCH // 16, srow, 0)

    for i in range(nchunk // _NC):
        for c in range(_NC):
            chunk = _NC * i + c

            @pl.when(core == c)
            def _():
                r0 = sub * acc_rows_per_tile
                pltpu.sync_copy(qinit.at[chunk].at[pl.ds(r0, acc_rows_per_tile)],
                                acc.at[pl.ds(r0, acc_rows_per_tile)])

        plsc.subcore_barrier()

        for c in range(_NC):
            chunk = _NC * i + c

            @pl.when(core == c)
            def _():
                tab = qtab.at[chunk]

                def sb_step(b, carry):
                    base = sub * rows_per_tile + b * _SBROWS
                    pltpu.sync_copy(esrc.at[pl.ds(base, _SBROWS)], sidx)
                    pltpu.sync_copy(escl.at[pl.ds(base, _SBROWS)], widx)
                    pltpu.sync_copy(edst.at[pl.ds(base, _SBROWS)], didx)
                    g = [None] * _SBROWS
                    w = [None] * _SBROWS
                    s = [None] * _SBROWS
                    for t in range(_SBROWS):
                        k = t % _NBUF
                        if t >= _NBUF:
                            s[t - _NBUF].wait()
                        g[t] = pltpu.async_copy(tab.at[sidx.at[t]],
                                                rows.at[k], sem_g.at[k])
                        w[t] = pltpu.async_copy(invsd.at[widx.at[t]],
                                                wbuf.at[k], sem_w.at[k])
                        r = t - (_NBUF - 1)
                        if r >= 0:
                            g[r].wait()
                            w[r].wait()
                            scale_rows(r % _NBUF)
                            s[r] = pltpu.async_copy(rows.at[r % _NBUF],
                                                    acc.at[didx.at[r]],
                                                    sem_s.at[r % _NBUF],
                                                    add=True)
                    for r in range(_SBROWS - _NBUF + 1, _SBROWS):
                        g[r].wait()
                        w[r].wait()
                        scale_rows(r % _NBUF)
                        s[r] = pltpu.async_copy(rows.at[r % _NBUF],
                                                acc.at[didx.at[r]],
                                                sem_s.at[r % _NBUF], add=True)
                    for r in range(_SBROWS - _NBUF, _SBROWS):
                        s[r].wait()
                    return carry

                lax.fori_loop(0, n_super, sb_step, 0)

        plsc.subcore_barrier()

        for c in range(_NC):
            chunk = _NC * i + c

            @pl.when(core == c)
            def _():
                r0 = sub * acc_rows_per_tile
                pltpu.sync_copy(acc.at[pl.ds(r0, acc_rows_per_tile)],
                                out.at[chunk].at[pl.ds(r0, acc_rows_per_tile)])

        plsc.subcore_barrier()


def _sc_qterm(qtab_chunks, invsd_flat, qinit, q2d, scl2d, dst2d):
    nchunk = qtab_chunks.shape[0]
    f = pl.kernel(
        _q_body,
        out_type=jax.ShapeDtypeStruct((nchunk, _ACCR, _CW), jnp.float32),
        mesh=plsc.VectorSubcoreMesh(core_axis_name="c", subcore_axis_name="s"),
        compiler_params=pltpu.CompilerParams(use_tc_tiling_on_sc=False),
        scratch_types=[
            pltpu.VMEM_SHARED((_ACCR, _CW), jnp.float32),
            pltpu.VMEM((_SBROWS, 128), jnp.int32),
            pltpu.VMEM((_SBROWS, 128), jnp.int32),
            pltpu.VMEM((_SBROWS, 128), jnp.int32),
            pltpu.VMEM((_NBUF, _EBATCH, _CW), jnp.float32),
            pltpu.VMEM((_NBUF, _EBATCH), jnp.float32),
            pltpu.SemaphoreType.DMA((_NBUF,)),
            pltpu.SemaphoreType.DMA((_NBUF,)),
            pltpu.SemaphoreType.DMA((_NBUF,)),
        ],
    )
    return f(qtab_chunks, invsd_flat, qinit, q2d, scl2d, dst2d)


def _pad_edges(arrays, fills):
    e = arrays[0].shape[0]
    unit = _NS * 2 * _SBROWS * 128
    epad = ((e + unit - 1) // unit) * unit
    pad = epad - e + _SBROWS * 128
    out = []
    for a, fill in zip(arrays, fills):
        a = jnp.concatenate([a.astype(jnp.int32),
                             jnp.full((pad,), fill, jnp.int32)])
        out.append(a.reshape(-1, 128))
    return out


def _mhsa_mean(x, Wq, Wk, Wv):
    B, S, D = x.shape
    h = D // HEADS
    q = (x @ Wq).reshape(B, S, HEADS, h).transpose(0, 2, 1, 3)
    k = (x @ Wk).reshape(B, S, HEADS, h).transpose(0, 2, 1, 3)
    v = (x @ Wv).reshape(B, S, HEADS, h).transpose(0, 2, 1, 3)
    att = jax.nn.softmax(jnp.einsum('bhqd,bhkd->bhqk', q, k) * (1.0 / float(np.sqrt(h))), axis=-1)
    return jnp.einsum('bhqk,bhkd->bhqd', att, v).transpose(0, 2, 1, 3).reshape(B, S, D).mean(axis=1)


def _score_body(pm_ref, cand_ref, out_ref):
    pm = pm_ref[...]
    cand = cand_ref[...]
    out_ref[...] = jnp.sum(pm[:, None, :] * cand, axis=-1)


def _chunked(x128):
    n = x128.shape[0]
    return jnp.moveaxis(x128.reshape(n, _NCHUNK, _CW), 1, 0)


def kernel(word_table, ent_table, d_Wq, d_Wk, d_Wv, t_Wq, t_Wk, t_Wv, W_t, b_t,
           users, items, query_words, neg_items, review_words, query_words_graph,
           profile_src, profile_dst, p_src, p_dst, q_id, pb_src, pb_dst):
    N, ED = ent_table.shape

    review_h = _mhsa_mean(word_table[review_words], d_Wq, d_Wk, d_Wv)
    deg_p = jnp.maximum(jax.ops.segment_sum(jnp.ones(profile_dst.shape[0], jnp.float32), profile_dst, N), 1.0)[:, None]
    psrc2d, pdst2d = _pad_edges((profile_src, profile_dst), (0, N))
    rh_chunks = jnp.moveaxis(review_h.reshape(-1, 2, _CW), 1, 0)
    zinit2 = jnp.zeros((2, _ACCR, _CW), jnp.float32)
    ph = _sc_segsum(rh_chunks, zinit2, psrc2d, pdst2d)
    entity_h = jnp.moveaxis(ph[:, :N, :], 0, 1).reshape(N, 2 * _CW) / deg_p

    qw = word_table[query_words_graph]
    q_l = _mhsa_mean(qw, t_Wq, t_Wk, t_Wv) @ W_t + b_t
    q_h = _mhsa_mean(qw, d_Wq, d_Wk, d_Wv)
    q_e0 = jnp.concatenate([q_l, q_h], axis=-1)

    e0 = jnp.concatenate([ent_table, entity_h], axis=-1)
    deg_i = jnp.maximum(
        jax.ops.segment_sum(jnp.ones(p_src.shape[0], jnp.float32), p_src, N)
        + jax.ops.segment_sum(jnp.ones(pb_src.shape[0], jnp.float32), pb_src, N), 1.0)[:, None]
    inv_sd = jax.lax.rsqrt(deg_i)

    invsd_flat = jnp.zeros((_ACCR,), jnp.float32).at[:N].set(inv_sd[:, 0])
    zinit = jnp.zeros((_NCHUNK, _ACCR, _CW), jnp.float32)
    q2d, scl2d, qdst2d = _pad_edges((q_id, p_src, p_dst), (0, 0, N))
    qinit = _sc_qterm(_chunked(q_e0), invsd_flat, zinit, q2d, scl2d, qdst2d)

    esrc2d, edst2d = _pad_edges(
        (jnp.concatenate([p_src, pb_src + N]),
         jnp.concatenate([p_dst, pb_dst])), (0, N))

    x = e0
    layers_sum = e0
    for _ in range(CONV):
        tab = _chunked(jnp.concatenate([x, x * inv_sd], axis=0))
        agg = _sc_segsum(tab, qinit, esrc2d, edst2d)
        agg = jnp.moveaxis(agg[:, :N, :], 0, 1).reshape(N, _NCHUNK * _CW)
        x = agg * inv_sd
        layers_sum = layers_sum + x
    ent_e = layers_sum * (1.0 / 3.0)

    user_emb = ent_e[users]
    qt = _mhsa_mean(word_table[query_words], t_Wq, t_Wk, t_Wv) @ W_t + b_t
    pm = user_emb[:, :ED] + qt
    cand = jnp.concatenate([ent_e[items][:, None, :ED], ent_e[neg_items][:, :, :ED]], axis=1)

    B = pm.shape[0]
    out = pl.pallas_call(
        _score_body,
        out_shape=jax.ShapeDtypeStruct((B, 6), jnp.float32),
    )(pm, cand)
    return out
